# Initial kernel scaffold; baseline (speedup 1.0000x reference)
import jax
import jax.numpy as jnp
from jax.experimental import pallas as pl


def kernel(world_pos, prev_world_pos, mesh_pos, node_type, cells, params):
    raise NotImplementedError("write your pallas kernel here")


# speedup vs baseline: 6.0338x; 6.0338x over previous
import functools

import jax
import jax.numpy as jnp
from jax import lax
from jax.experimental import pallas as pl
from jax.experimental.pallas import tpu as pltpu
from jax.experimental.pallas import tpu_sc as plsc

N = 10000
E = 160002
D = 128
NT = 9

NC = 2
NS = 16
NW = NC * NS
CH = 128

E_PAD = 163840
N_PAD = 10240
GCH = (2 * E_PAD) // (NW * CH)
SCH = E_PAD // (NW * CH)
ROWS_PER_SUB = N_PAD // NS

BLK_E = 2048
BLK_N = 2048


def _make_gather(width, n_out_rows, chunks_per_worker):
    mesh = plsc.VectorSubcoreMesh(core_axis_name="c", subcore_axis_name="s",
                                  num_cores=NC, num_subcores=NS)
    rows_per_worker = chunks_per_worker * CH

    @functools.partial(
        pl.kernel,
        mesh=mesh,
        out_type=jax.ShapeDtypeStruct((n_out_rows, width), jnp.float32),
        scratch_types=[
            pltpu.VMEM((chunks_per_worker, CH), jnp.int32),
            pltpu.VMEM((CH, width), jnp.float32),
            pltpu.VMEM((CH, width), jnp.float32),
            pltpu.SemaphoreType.DMA,
            pltpu.SemaphoreType.DMA,
        ],
    )
    def gather_k(table, idx3, out, idx_v, buf0, buf1, sem0, sem1):
        w = lax.axis_index("c") * NS + lax.axis_index("s")
        base = w * rows_per_worker
        pltpu.sync_copy(idx3.at[w], idx_v)

        def body(j, carry):
            i0 = 2 * j
            i1 = i0 + 1
            c0 = pltpu.make_async_copy(table.at[idx_v.at[i0]], buf0, sem0)
            c1 = pltpu.make_async_copy(table.at[idx_v.at[i1]], buf1, sem1)
            c0.start()
            c1.start()
            c0.wait()
            pltpu.sync_copy(buf0, out.at[pl.ds(base + i0 * CH, CH)])
            c1.wait()
            pltpu.sync_copy(buf1, out.at[pl.ds(base + i1 * CH, CH)])
            return carry

        lax.fori_loop(0, chunks_per_worker // 2, body, 0)

    return gather_k


@functools.cache
def _gather_kernel(width, n_out_rows, chunks_per_worker):
    return _make_gather(width, n_out_rows, chunks_per_worker)


def _gather_latent(table, idx3):
    return _gather_kernel(D, 2 * E_PAD, GCH)(table, idx3)


def _gather_pos(table, idx3):
    return _gather_kernel(D, 2 * E_PAD, GCH)(table, idx3)


@functools.cache
def _scatter_kernel():
    mesh = plsc.VectorSubcoreMesh(core_axis_name="c", subcore_axis_name="s",
                                  num_cores=NC, num_subcores=NS)

    @functools.partial(
        pl.kernel,
        mesh=mesh,
        out_type=jax.ShapeDtypeStruct((2, N_PAD, D), jnp.float32),
        scratch_types=[
            pltpu.VMEM((SCH, CH), jnp.int32),
            pltpu.VMEM((CH, D), jnp.float32),
            pltpu.VMEM_SHARED((N_PAD, D), jnp.float32),
            pltpu.SemaphoreType.DMA,
        ],
    )
    def scatter_k(edges, recv3, zeros_hbm, out, idx_v, ebuf, acc, sem):
        c = lax.axis_index("c")
        s = lax.axis_index("s")
        w = c * NS + s
        pltpu.sync_copy(zeros_hbm.at[pl.ds(s * ROWS_PER_SUB, ROWS_PER_SUB)],
                        acc.at[pl.ds(s * ROWS_PER_SUB, ROWS_PER_SUB)])
        pltpu.sync_copy(recv3.at[w], idx_v)
        plsc.subcore_barrier()

        base = w * SCH * CH

        def body(i, carry):
            pltpu.sync_copy(edges.at[pl.ds(base + i * CH, CH)], ebuf)
            pltpu.sync_copy(ebuf, acc.at[idx_v.at[i]], add=True)
            return carry

        lax.fori_loop(0, SCH, body, 0)
        plsc.subcore_barrier()
        pltpu.sync_copy(acc.at[pl.ds(s * ROWS_PER_SUB, ROWS_PER_SUB)],
                        out.at[c, pl.ds(s * ROWS_PER_SUB, ROWS_PER_SUB)])

    return scatter_k


def _scatter_add(edges, recv3, zacc):
    return _scatter_kernel()(edges, recv3, zacc)


def _ln(h, g, b):
    m = jnp.mean(h, axis=-1, keepdims=True)
    v = jnp.mean((h - m) * (h - m), axis=-1, keepdims=True)
    return (h - m) * lax.rsqrt(v + 1e-5) * g + b


def _row_spec(blk, width):
    return pl.BlockSpec((blk, width), lambda i: (i, 0))


def _full_spec(shape):
    return pl.BlockSpec(shape, lambda i: (0,) * len(shape))


def _tc_call(body, grid, in_specs, out_rows, blk):
    return pl.pallas_call(
        body,
        grid=(grid,),
        in_specs=in_specs,
        out_specs=_row_spec(blk, D),
        out_shape=jax.ShapeDtypeStruct((out_rows, D), jnp.float32),
        compiler_params=pltpu.CompilerParams(
            dimension_semantics=("arbitrary",)),
    )


def _edge_enc_body(ps_ref, pr_ref, mw_ref, mm_ref, wd_ref, wv_ref, mv_ref,
                   c1_ref, w2_ref, b2_ref, w3_ref, b3_ref, g_ref, bl_ref,
                   o_ref):
    i = pl.program_id(0)
    d = ps_ref[...] - pr_ref[...]
    dd = d * d
    wn = jnp.sqrt(jnp.sum(dd * mw_ref[...], axis=-1, keepdims=True))
    mn = jnp.sqrt(jnp.sum(dd * mm_ref[...], axis=-1, keepdims=True))
    h = (jnp.dot(d, wd_ref[...], preferred_element_type=jnp.float32)
         + wn * wv_ref[...] + mn * mv_ref[...] + c1_ref[...])
    h = jnp.maximum(h, 0.0)
    h = jnp.maximum(jnp.dot(h, w2_ref[...],
                            preferred_element_type=jnp.float32) + b2_ref[...],
                    0.0)
    h = jnp.dot(h, w3_ref[...], preferred_element_type=jnp.float32) + b3_ref[...]
    h = _ln(h, g_ref[...], bl_ref[...])
    rows = i * BLK_E + lax.broadcasted_iota(jnp.int32, (BLK_E, 1), 0)
    o_ref[...] = jnp.where(rows < E, h, 0.0)


def _node_enc_body(f_ref, w1_ref, b1_ref, w2_ref, b2_ref, w3_ref, b3_ref,
                   g_ref, bl_ref, o_ref):
    h = jnp.dot(f_ref[...], w1_ref[...],
                preferred_element_type=jnp.float32) + b1_ref[...]
    h = jnp.maximum(h, 0.0)
    h = jnp.maximum(jnp.dot(h, w2_ref[...],
                            preferred_element_type=jnp.float32) + b2_ref[...],
                    0.0)
    h = jnp.dot(h, w3_ref[...], preferred_element_type=jnp.float32) + b3_ref[...]
    o_ref[...] = _ln(h, g_ref[...], bl_ref[...])


def _edge_step_body(e_ref, s_ref, r_ref, w1a_ref, w1b_ref, w1c_ref, b1_ref,
                    w2_ref, b2_ref, w3_ref, b3_ref, g_ref, bl_ref, o_ref):
    i = pl.program_id(0)
    x = e_ref[...]
    h = (jnp.dot(x, w1a_ref[...], preferred_element_type=jnp.float32)
         + jnp.dot(s_ref[...], w1b_ref[...], preferred_element_type=jnp.float32)
         + jnp.dot(r_ref[...], w1c_ref[...], preferred_element_type=jnp.float32)
         + b1_ref[...])
    h = jnp.maximum(h, 0.0)
    h = jnp.maximum(jnp.dot(h, w2_ref[...],
                            preferred_element_type=jnp.float32) + b2_ref[...],
                    0.0)
    h = jnp.dot(h, w3_ref[...], preferred_element_type=jnp.float32) + b3_ref[...]
    out = x + _ln(h, g_ref[...], bl_ref[...])
    rows = i * BLK_E + lax.broadcasted_iota(jnp.int32, (BLK_E, 1), 0)
    o_ref[...] = jnp.where(rows < E, out, 0.0)


def _node_step_body(n_ref, p0_ref, p1_ref, w1a_ref, w1b_ref, b1_ref, w2_ref,
                    b2_ref, w3_ref, b3_ref, g_ref, bl_ref, o_ref):
    x = n_ref[...]
    agg = p0_ref[0] + p1_ref[0]
    h = (jnp.dot(x, w1a_ref[...], preferred_element_type=jnp.float32)
         + jnp.dot(agg, w1b_ref[...], preferred_element_type=jnp.float32)
         + b1_ref[...])
    h = jnp.maximum(h, 0.0)
    h = jnp.maximum(jnp.dot(h, w2_ref[...],
                            preferred_element_type=jnp.float32) + b2_ref[...],
                    0.0)
    h = jnp.dot(h, w3_ref[...], preferred_element_type=jnp.float32) + b3_ref[...]
    o_ref[...] = x + _ln(h, g_ref[...], bl_ref[...])


def _decoder_body(n_ref, w1_ref, b1_ref, w2_ref, b2_ref, w3_ref, b3_ref,
                  o_ref):
    h = jnp.dot(n_ref[...], w1_ref[...],
                preferred_element_type=jnp.float32) + b1_ref[...]
    h = jnp.maximum(h, 0.0)
    h = jnp.maximum(jnp.dot(h, w2_ref[...],
                            preferred_element_type=jnp.float32) + b2_ref[...],
                    0.0)
    o_ref[...] = jnp.dot(h, w3_ref[...],
                         preferred_element_type=jnp.float32) + b3_ref[...]


def _r2(x):
    return x.reshape(1, -1)


def kernel(world_pos, prev_world_pos, mesh_pos, node_type, cells, params):
    wp = world_pos[0]
    pwp = prev_world_pos[0]
    mp = mesh_pos[0]

    c0 = cells[0].astype(jnp.int32)
    senders = jnp.concatenate([c0[:, 0], c0[:, 1], c0[:, 2],
                               c0[:, 1], c0[:, 2], c0[:, 0]])
    receivers = jnp.concatenate([c0[:, 1], c0[:, 2], c0[:, 0],
                                 c0[:, 0], c0[:, 1], c0[:, 2]])
    sp = jnp.zeros((E_PAD,), jnp.int32).at[:E].set(senders)
    rp = jnp.zeros((E_PAD,), jnp.int32).at[:E].set(receivers)
    idx3 = jnp.concatenate([sp, rp]).reshape(NW, GCH, CH)
    recv3 = rp.reshape(NW, SCH, CH)
    zacc = jnp.zeros((N_PAD, D), jnp.float32)

    pos_tab = jnp.zeros((N_PAD, D), jnp.float32)
    pos_tab = pos_tab.at[:N, 0:3].set(wp).at[:N, 3:5].set(mp)

    vel = wp - pwp
    one_hot = jax.nn.one_hot(node_type[0, :, 0], NT, dtype=jnp.float32)
    nmu, nsig = params['node_norm']
    nf = (jnp.concatenate([vel, one_hot], axis=-1) - nmu) / nsig
    nf16 = jnp.zeros((N_PAD, 16), jnp.float32).at[:N, :12].set(nf)

    emu, esig = params['edge_norm']
    (eW1, eb1), (eW2, eb2), (eW3, eb3) = params['edge_enc']['mlp']
    eg, ebl = params['edge_enc']['ln']
    W1s = eW1 / esig[:, None]
    c1 = eb1 - emu @ W1s
    Wd = jnp.zeros((D, D), jnp.float32).at[0:3].set(W1s[0:3]).at[3:5].set(W1s[4:6])
    wvec = W1s[3:4]
    mvec = W1s[6:7]
    maskw = jnp.zeros((1, D), jnp.float32).at[0, 0:3].set(1.0)
    maskm = jnp.zeros((1, D), jnp.float32).at[0, 3:5].set(1.0)

    P = _gather_pos(pos_tab, idx3)

    ge = E_PAD // BLK_E
    n_off = E_PAD // BLK_E
    edge_in_specs = [
        pl.BlockSpec((BLK_E, D), lambda i: (i, 0)),
        pl.BlockSpec((BLK_E, D), lambda i: (i + n_off, 0)),
        _full_spec((1, D)), _full_spec((1, D)),
        _full_spec((D, D)), _full_spec((1, D)), _full_spec((1, D)),
        _full_spec((1, D)), _full_spec((D, D)), _full_spec((1, D)),
        _full_spec((D, D)), _full_spec((1, D)),
        _full_spec((1, D)), _full_spec((1, D)),
    ]
    edge_latent = _tc_call(_edge_enc_body, ge, edge_in_specs, E_PAD, BLK_E)(
        P, P, maskw, maskm, Wd, wvec, mvec, _r2(c1), eW2, _r2(eb2),
        eW3, _r2(eb3), _r2(eg), _r2(ebl))

    (nW1, nb1), (nW2, nb2), (nW3, nb3) = params['node_enc']['mlp']
    ng, nbl = params['node_enc']['ln']
    nW1p = jnp.zeros((16, D), jnp.float32).at[:12].set(nW1)
    gn = N_PAD // BLK_N
    node_in_specs = [
        _row_spec(BLK_N, 16),
        _full_spec((16, D)), _full_spec((1, D)),
        _full_spec((D, D)), _full_spec((1, D)),
        _full_spec((D, D)), _full_spec((1, D)),
        _full_spec((1, D)), _full_spec((1, D)),
    ]
    node_in_specs[0] = pl.BlockSpec((BLK_N, 16), lambda i: (i, 0))
    node_latent = _tc_call(_node_enc_body, gn, node_in_specs, N_PAD, BLK_N)(
        nf16, nW1p, _r2(nb1), nW2, _r2(nb2), nW3, _r2(nb3), _r2(ng), _r2(nbl))

    estep_specs = [
        _row_spec(BLK_E, D),
        pl.BlockSpec((BLK_E, D), lambda i: (i, 0)),
        pl.BlockSpec((BLK_E, D), lambda i: (i + n_off, 0)),
        _full_spec((D, D)), _full_spec((D, D)), _full_spec((D, D)),
        _full_spec((1, D)),
        _full_spec((D, D)), _full_spec((1, D)),
        _full_spec((D, D)), _full_spec((1, D)),
        _full_spec((1, D)), _full_spec((1, D)),
    ]
    nstep_specs = [
        _row_spec(BLK_N, D),
        pl.BlockSpec((1, BLK_N, D), lambda i: (0, i, 0)),
        pl.BlockSpec((1, BLK_N, D), lambda i: (1, i, 0)),
        _full_spec((D, D)), _full_spec((D, D)),
        _full_spec((1, D)),
        _full_spec((D, D)), _full_spec((1, D)),
        _full_spec((D, D)), _full_spec((1, D)),
        _full_spec((1, D)), _full_spec((1, D)),
    ]

    for step in params['steps']:
        (sW1, sb1), (sW2, sb2), (sW3, sb3) = step['edge']['mlp']
        sg, sbl = step['edge']['ln']
        G = _gather_latent(node_latent, idx3)
        edge_latent = _tc_call(_edge_step_body, ge, estep_specs, E_PAD, BLK_E)(
            edge_latent, G, G,
            sW1[0:D], sW1[D:2 * D], sW1[2 * D:3 * D], _r2(sb1),
            sW2, _r2(sb2), sW3, _r2(sb3), _r2(sg), _r2(sbl))

        partials = _scatter_add(edge_latent, recv3, zacc)

        (tW1, tb1), (tW2, tb2), (tW3, tb3) = step['node']['mlp']
        tg, tbl = step['node']['ln']
        node_latent = _tc_call(_node_step_body, gn, nstep_specs, N_PAD, BLK_N)(
            node_latent, partials, partials,
            tW1[0:D], tW1[D:2 * D], _r2(tb1),
            tW2, _r2(tb2), tW3, _r2(tb3), _r2(tg), _r2(tbl))

    (dW1, db1), (dW2, db2), (dW3, db3) = params['decoder']
    dW3p = jnp.zeros((D, D), jnp.float32).at[:, 0:3].set(dW3)
    db3p = jnp.zeros((D,), jnp.float32).at[0:3].set(db3)
    dec_specs = [
        _row_spec(BLK_N, D),
        _full_spec((D, D)), _full_spec((1, D)),
        _full_spec((D, D)), _full_spec((1, D)),
        _full_spec((D, D)), _full_spec((1, D)),
    ]
    net = _tc_call(_decoder_body, gn, dec_specs, N_PAD, BLK_N)(
        node_latent, dW1, _r2(db1), dW2, _r2(db2), dW3p, _r2(db3p))

    net3 = net[:N, 0:3]
    on_mu, on_sig = params['out_norm']
    acceleration = net3 * on_sig + on_mu
    return (2 * wp + acceleration - pwp)[None]
